# Initial kernel scaffold; baseline (speedup 1.0000x reference)
import jax
import jax.numpy as jnp
from jax.experimental import pallas as pl


def kernel(x, edge_index, edge_attr, batch, W1, R1, b1, W2, R2, b2, W3, R3, b3):
    raise NotImplementedError("write your pallas kernel here")


# speedup vs baseline: 3.1139x; 3.1139x over previous
import functools

import jax
import jax.numpy as jnp
from jax import lax
from jax.experimental import pallas as pl
from jax.experimental.pallas import tpu as pltpu
from jax.experimental.pallas import tpu_sc as plsc

NN = 10000
EE = 320000
FF = 128
CC = 10
GG = 8

NPAD = 10240
ECHUNK = 128
NSUB = 16
NCORE = 2
CHUNKS_PER_TILE = 157
EPAD = CHUNKS_PER_TILE * NSUB * ECHUNK
ROWS_PER_TILE = NPAD // NSUB
FH = FF // 2


def _sc_mesh():
    return plsc.VectorSubcoreMesh(core_axis_name="c", subcore_axis_name="s")


def _sc_pass(weighted: bool, with_cnt: bool):
    out_types = [jax.ShapeDtypeStruct((NCORE * NPAD, FH), jnp.float32)]
    if weighted:
        out_types.append(jax.ShapeDtypeStruct((NCORE * NPAD, FH), jnp.float32))
    if with_cnt:
        out_types.append(jax.ShapeDtypeStruct((NPAD, 16), jnp.float32))

    scratch = [
        pltpu.VMEM_SHARED((NPAD, FH), jnp.float32),
        pltpu.VMEM((ECHUNK,), jnp.int32),
        pltpu.VMEM((ECHUNK,), jnp.int32),
        pltpu.VMEM((ECHUNK,), jnp.int32),
        pltpu.VMEM((ECHUNK, FH), jnp.float32),
        pltpu.SemaphoreType.DMA,
    ]
    if weighted:
        scratch.insert(1, pltpu.VMEM_SHARED((NPAD, FH), jnp.float32))
        scratch.append(pltpu.VMEM((ECHUNK, 16), jnp.float32))
        scratch.append(pltpu.VMEM((ECHUNK, FH), jnp.float32))
    if with_cnt:
        scratch.append(pltpu.VMEM_SHARED((NPAD, 16), jnp.float32))
        scratch.append(pltpu.VMEM((ECHUNK, 16), jnp.float32))

    def body(*refs):
        it = iter(refs)
        h_hbm = next(it)
        src_hbm = next(it)
        dst_hbm = next(it)
        u_hbm = next(it) if weighted else None
        z64_hbm = next(it)
        z16_hbm = next(it) if with_cnt else None
        ones_hbm = next(it) if with_cnt else None
        accS_out = next(it)
        acc1_out = next(it) if weighted else None
        cnt_out = next(it) if with_cnt else None
        accS_sh = next(it)
        acc1_sh = next(it) if weighted else None
        srcbuf = next(it)
        idxbuf = next(it)
        dstbuf = next(it)
        gbuf = next(it)
        sem = next(it)
        if weighted:
            ubuf = next(it)
            wbuf = next(it)
        if with_cnt:
            cnt_sh = next(it)
            onesbuf = next(it)

        c = lax.axis_index("c")
        s = lax.axis_index("s")
        rbase = s * ROWS_PER_TILE

        pltpu.sync_copy(z64_hbm.at[pl.ds(rbase, ROWS_PER_TILE)],
                        accS_sh.at[pl.ds(rbase, ROWS_PER_TILE)])
        if weighted:
            pltpu.sync_copy(z64_hbm.at[pl.ds(rbase, ROWS_PER_TILE)],
                            acc1_sh.at[pl.ds(rbase, ROWS_PER_TILE)])
        if with_cnt:
            @pl.when(c == 0)
            def _():
                pltpu.sync_copy(z16_hbm.at[pl.ds(rbase, ROWS_PER_TILE)],
                                cnt_sh.at[pl.ds(rbase, ROWS_PER_TILE)])
                pltpu.sync_copy(ones_hbm, onesbuf)
        plsc.subcore_barrier()

        def chunk(j, carry):
            base = (s * CHUNKS_PER_TILE + j) * ECHUNK
            pltpu.sync_copy(src_hbm.at[pl.ds(base, ECHUNK)], srcbuf)
            pltpu.sync_copy(dst_hbm.at[pl.ds(base, ECHUNK)], dstbuf)
            if weighted:
                pltpu.sync_copy(u_hbm.at[pl.ds(base, ECHUNK)], ubuf)
            for g in range(ECHUNK // 16):
                sv = srcbuf[pl.ds(g * 16, 16)]
                idxbuf[pl.ds(g * 16, 16)] = sv * 2 + c
            pltpu.async_copy(h_hbm.at[idxbuf], gbuf, sem).wait()
            if weighted:
                for e in range(ECHUNK):
                    ue = ubuf[e, :]
                    for fb in range(FH // 16):
                        wbuf[e, pl.ds(fb * 16, 16)] = (
                            gbuf[e, pl.ds(fb * 16, 16)] * ue)
            pltpu.sync_copy(gbuf, accS_sh.at[dstbuf], add=True)
            if weighted:
                pltpu.sync_copy(wbuf, acc1_sh.at[dstbuf], add=True)
            if with_cnt:
                @pl.when(c == 0)
                def _():
                    pltpu.sync_copy(onesbuf, cnt_sh.at[dstbuf], add=True)
            return carry

        lax.fori_loop(0, CHUNKS_PER_TILE, chunk, 0)
        plsc.subcore_barrier()

        obase = c * NPAD + rbase
        pltpu.sync_copy(accS_sh.at[pl.ds(rbase, ROWS_PER_TILE)],
                        accS_out.at[pl.ds(obase, ROWS_PER_TILE)])
        if weighted:
            pltpu.sync_copy(acc1_sh.at[pl.ds(rbase, ROWS_PER_TILE)],
                            acc1_out.at[pl.ds(obase, ROWS_PER_TILE)])
        if with_cnt:
            @pl.when(c == 0)
            def _():
                pltpu.sync_copy(cnt_sh.at[pl.ds(rbase, ROWS_PER_TILE)],
                                cnt_out.at[pl.ds(rbase, ROWS_PER_TILE)])

    return pl.kernel(body, out_type=tuple(out_types), mesh=_sc_mesh(),
                     scratch_types=scratch,
                     compiler_params=pltpu.CompilerParams(
                         use_tc_tiling_on_sc=False))


BR = 512


def _tc_layer_body(accS_ref, acc1_ref, cnt_ref, h_ref, W_ref, R_ref, b_ref,
                   out_ref):
    W0 = W_ref[0]
    Wd = W_ref[1] - W_ref[0]
    aS = jnp.concatenate([accS_ref[0], accS_ref[1]], axis=1)
    a1 = jnp.concatenate([acc1_ref[0], acc1_ref[1]], axis=1)
    agg = (jnp.dot(aS, W0, preferred_element_type=jnp.float32)
           + jnp.dot(a1, Wd, preferred_element_type=jnp.float32))
    rc = 1.0 / jnp.maximum(cnt_ref[:, 0:1], 1.0)
    h = (agg * rc
         + jnp.dot(h_ref[...], R_ref[...], preferred_element_type=jnp.float32)
         + b_ref[...])
    out_ref[...] = h * jax.nn.sigmoid(h)


def _tc_layer(accS, acc1, cnt16, h_prev, W, R, b_row):
    nblk = NPAD // BR
    return pl.pallas_call(
        _tc_layer_body,
        grid=(nblk,),
        in_specs=[
            pl.BlockSpec((2, BR, FH), lambda i: (0, i, 0)),
            pl.BlockSpec((2, BR, FH), lambda i: (0, i, 0)),
            pl.BlockSpec((BR, 16), lambda i: (i, 0)),
            pl.BlockSpec((BR, FF), lambda i: (i, 0)),
            pl.BlockSpec((2, FF, FF), lambda i: (0, 0, 0)),
            pl.BlockSpec((FF, FF), lambda i: (0, 0)),
            pl.BlockSpec((1, FF), lambda i: (0, 0)),
        ],
        out_specs=pl.BlockSpec((BR, FF), lambda i: (i, 0)),
        out_shape=jax.ShapeDtypeStruct((NPAD, FF), jnp.float32),
    )(accS, acc1, cnt16, h_prev, W, R, b_row)


def _tc_final_body(accS_ref, cnt_ref, h_ref, batch_ref, W_ref, R_ref, b_ref,
                   out_ref, gmp_acc, sum_acc, cg_acc):
    i = pl.program_id(0)
    nblk = pl.num_programs(0)

    @pl.when(i == 0)
    def _():
        gmp_acc[...] = jnp.full((GG, FF), -jnp.inf, jnp.float32)
        sum_acc[...] = jnp.zeros((GG, FF), jnp.float32)
        cg_acc[...] = jnp.zeros((GG, FF), jnp.float32)

    aS = jnp.concatenate([accS_ref[0], accS_ref[1]], axis=1)
    agg = jnp.dot(aS, W_ref[...], preferred_element_type=jnp.float32)
    rc = 1.0 / jnp.maximum(cnt_ref[:, 0:1], 1.0)
    h3 = (agg * rc
          + jnp.dot(h_ref[...], R_ref[...], preferred_element_type=jnp.float32)
          + b_ref[...])

    bvec = batch_ref[...]
    gidx = lax.broadcasted_iota(jnp.int32, (BR, GG), 1)
    mask = (bvec == gidx).astype(jnp.float32)
    sum_acc[...] += jnp.dot(mask.T, h3, preferred_element_type=jnp.float32)
    cg_acc[...] += jnp.dot(mask.T, jnp.ones_like(h3),
                           preferred_element_type=jnp.float32)
    for g in range(GG):
        mg = bvec == g
        colmax = jnp.max(jnp.where(mg, h3, -jnp.inf), axis=0, keepdims=True)
        gmp_acc[g:g + 1, :] = jnp.maximum(gmp_acc[g:g + 1, :], colmax)

    @pl.when(i == nblk - 1)
    def _():
        gap = sum_acc[...] / jnp.maximum(cg_acc[...], 1.0)
        o = jnp.concatenate([gmp_acc[:, :CC], gap[:, :CC]], axis=1)
        m = jnp.max(o, axis=1, keepdims=True)
        z = o - m
        lse = jnp.log(jnp.sum(jnp.exp(z), axis=1, keepdims=True))
        out_ref[...] = z - lse


def _tc_final(accS, cnt16, h_prev, batch2d, W_col, R_col, b_row):
    nblk = NPAD // BR
    return pl.pallas_call(
        _tc_final_body,
        grid=(nblk,),
        in_specs=[
            pl.BlockSpec((2, BR, FH), lambda i: (0, i, 0)),
            pl.BlockSpec((BR, 16), lambda i: (i, 0)),
            pl.BlockSpec((BR, FF), lambda i: (i, 0)),
            pl.BlockSpec((BR, 1), lambda i: (i, 0)),
            pl.BlockSpec((FF, FF), lambda i: (0, 0)),
            pl.BlockSpec((FF, FF), lambda i: (0, 0)),
            pl.BlockSpec((1, FF), lambda i: (0, 0)),
        ],
        out_specs=pl.BlockSpec((GG, 2 * CC), lambda i: (0, 0)),
        out_shape=jax.ShapeDtypeStruct((GG, 2 * CC), jnp.float32),
        scratch_shapes=[
            pltpu.VMEM((GG, FF), jnp.float32),
            pltpu.VMEM((GG, FF), jnp.float32),
            pltpu.VMEM((GG, FF), jnp.float32),
        ],
        compiler_params=pltpu.CompilerParams(
            dimension_semantics=("arbitrary",)),
    )(accS, cnt16, h_prev, batch2d, W_col, R_col, b_row)


def kernel(x, edge_index, edge_attr, batch, W1, R1, b1, W2, R2, b2, W3, R3, b3):
    src = edge_index[0]
    dst = edge_index[1]
    u = edge_attr[:, 0]

    srcp = jnp.pad(src, (0, EPAD - EE))
    dstp = jnp.pad(dst, (0, EPAD - EE), constant_values=NN)
    up = jnp.pad(u, (0, EPAD - EE))
    ubig = jnp.broadcast_to(up[:, None], (EPAD, 16))

    xp = jnp.pad(x, ((0, NPAD - NN), (0, 0)))
    batchp = jnp.pad(batch, (0, NPAD - NN), constant_values=GG)
    batch2d = batchp.reshape(NPAD, 1)

    z64 = jnp.zeros((NPAD, FH), jnp.float32)
    z16 = jnp.zeros((NPAD, 16), jnp.float32)
    ones16 = jnp.ones((ECHUNK, 16), jnp.float32)

    b1r = b1.reshape(1, FF)
    b2r = b2.reshape(1, FF)
    W3p = jnp.pad(W3[0], ((0, 0), (0, FF - CC)))
    R3p = jnp.pad(R3, ((0, 0), (0, FF - CC)))
    b3r = jnp.pad(b3, (0, FF - CC)).reshape(1, FF)

    sc1 = _sc_pass(weighted=True, with_cnt=True)
    sc2 = _sc_pass(weighted=True, with_cnt=False)
    sc3 = _sc_pass(weighted=False, with_cnt=False)

    accS, acc1, cnt16 = sc1(xp.reshape(2 * NPAD, FH), srcp, dstp, ubig,
                            z64, z16, ones16)
    h1 = _tc_layer(accS.reshape(2, NPAD, FH), acc1.reshape(2, NPAD, FH),
                   cnt16, xp, W1, R1, b1r)

    accS2, acc12 = sc2(h1.reshape(2 * NPAD, FH), srcp, dstp, ubig, z64)
    h2 = _tc_layer(accS2.reshape(2, NPAD, FH), acc12.reshape(2, NPAD, FH),
                   cnt16, h1, W2, R2, b2r)

    accS3, = sc3(h2.reshape(2 * NPAD, FH), srcp, dstp, z64)
    out = _tc_final(accS3.reshape(2, NPAD, FH), cnt16, h2, batch2d,
                    W3p, R3p, b3r)
    return out
